# baseline (speedup 1.0000x reference)
import functools

import jax
import jax.numpy as jnp
from jax.experimental import pallas as pl
from jax.experimental.pallas import tpu as pltpu


def _fused_se_kernel(x_ref, w1t_ref, b1_ref, a1_ref,
                     w2t_ref, b2_ref, a2_ref,
                     w3t_ref, b3_ref,
                     out_ref, *, inv_hw):
    x = x_ref[...]
    gap = jnp.sum(x.astype(jnp.float32), axis=(-2, -1)) * inv_hw
    y = jnp.dot(gap, w1t_ref[...], preferred_element_type=jnp.float32) + b1_ref[...]
    y = jnp.where(y >= 0.0, y, a1_ref[...] * y)
    y = jnp.dot(y, w2t_ref[...], preferred_element_type=jnp.float32) + b2_ref[...]
    y = jnp.where(y >= 0.0, y, a2_ref[...] * y)
    y = jnp.dot(y, w3t_ref[...], preferred_element_type=jnp.float32) + b3_ref[...]
    gate = jax.nn.sigmoid(y).astype(x.dtype)
    out_ref[...] = x * gate[:, :, None, None]


@jax.jit
def kernel(x, w1, b1, a1, w2, b2, a2, w3, b3):
    B, C, H, W = x.shape

    tb = 1

    w1t = w1.T
    w2t = w2.T
    w3t = w3.T

    full2 = lambda shape: pl.BlockSpec(shape, lambda i: (0, 0))

    body = functools.partial(_fused_se_kernel, inv_hw=1.0 / float(H * W))

    out = pl.pallas_call(
        body,
        out_shape=jax.ShapeDtypeStruct((B, C, H, W), x.dtype),
        grid=(B // tb,),
        in_specs=[
            pl.BlockSpec((tb, C, H, W), lambda i: (i, 0, 0, 0)),
            full2((C, C)), full2((1, C)), full2((1, C)),
            full2((C, C)), full2((1, C)), full2((1, C)),
            full2((C, C)), full2((1, C)),
        ],
        out_specs=pl.BlockSpec((tb, C, H, W), lambda i: (i, 0, 0, 0)),
        compiler_params=pltpu.CompilerParams(
            dimension_semantics=("parallel",),
            vmem_limit_bytes=56 * 2**20,
        ),
    )(
        x,
        w1t, b1, a1,
        w2t, b2, a2,
        w3t, b3,
    )
    return out


# speedup vs baseline: 10.6847x; 10.6847x over previous
import functools

import jax
import jax.numpy as jnp
from jax.experimental import pallas as pl
from jax.experimental.pallas import tpu as pltpu


def _fused_se_kernel(x_ref, w1t_ref, b1_ref, a1_ref,
                     w2t_ref, b2_ref, a2_ref,
                     w3t_ref, b3_ref,
                     out_ref, *, inv_hw):
    x = x_ref[...]
    gap = jnp.sum(x.astype(jnp.float32), axis=1) * inv_hw
    y = jnp.dot(gap, w1t_ref[...], preferred_element_type=jnp.float32) + b1_ref[...]
    y = jnp.where(y >= 0.0, y, a1_ref[...] * y)
    y = jnp.dot(y, w2t_ref[...], preferred_element_type=jnp.float32) + b2_ref[...]
    y = jnp.where(y >= 0.0, y, a2_ref[...] * y)
    y = jnp.dot(y, w3t_ref[...], preferred_element_type=jnp.float32) + b3_ref[...]
    gate = jax.nn.sigmoid(y).astype(x.dtype)
    out_ref[...] = x * gate[:, None, :]


@jax.jit
def kernel(x, w1, b1, a1, w2, b2, a2, w3, b3):
    B, C, H, W = x.shape
    hw = H * W

    xt = jnp.transpose(x, (0, 2, 3, 1)).reshape(B, hw, C)

    tb = 4
    while B % tb:
        tb //= 2

    w1t = w1.T
    w2t = w2.T
    w3t = w3.T

    full2 = lambda shape: pl.BlockSpec(shape, lambda i: (0, 0))

    tile_bytes = tb * hw * C * 4
    weight_bytes = 3 * C * C * 4 + 5 * C * 4
    vmem_limit = int(min(100 * 2**20, 4 * tile_bytes + 2 * weight_bytes + 2**20))

    body = functools.partial(_fused_se_kernel, inv_hw=1.0 / float(hw))

    outt = pl.pallas_call(
        body,
        out_shape=jax.ShapeDtypeStruct((B, hw, C), x.dtype),
        grid=(B // tb,),
        in_specs=[
            pl.BlockSpec((tb, hw, C), lambda i: (i, 0, 0)),
            full2((C, C)), full2((1, C)), full2((1, C)),
            full2((C, C)), full2((1, C)), full2((1, C)),
            full2((C, C)), full2((1, C)),
        ],
        out_specs=pl.BlockSpec((tb, hw, C), lambda i: (i, 0, 0)),
        compiler_params=pltpu.CompilerParams(
            dimension_semantics=("parallel",),
            vmem_limit_bytes=vmem_limit,
        ),
    )(
        xt,
        w1t, b1, a1,
        w2t, b2, a2,
        w3t, b3,
    )
    return outt.reshape(B, H, W, C).transpose(0, 3, 1, 2)
